# baseline (device time: 119506 ns/iter reference)
import jax
import jax.numpy as jnp
import math
from jax import lax
from jax.experimental import pallas as pl
from jax.experimental.pallas import tpu as pltpu

N_DEV = 4
S_PER = 1024
D_MODEL = 1024
HQ = 8
DH = 128
BLK = 64
SCALE = 0.08838834764831843
Q_SCALE = SCALE * math.log2(math.e)
NEG = -1e9
HALF = S_PER // 2


def kernel(x, Wq, K_ext, V_ext, Wo):
    x2 = x.reshape(S_PER, D_MODEL)
    k2 = K_ext.reshape(S_PER, HQ * DH).astype(jnp.bfloat16)
    v2 = V_ext.reshape(S_PER, HQ * DH).astype(jnp.bfloat16)

    def body(x_ref, wq_ref, k_ref, v_ref, wo_ref, out_ref,
             k_all, v_all, q_buf, bias_buf, acc_buf, ctx_buf, lsum_buf,
             kf_send, kf_recv, vf_send, vf_recv,
             kb_send, kb_recv, vb_send, vb_recv):
        my = lax.axis_index("i")
        left = lax.rem(my + (N_DEV - 1), N_DEV)
        right = lax.rem(my + 1, N_DEV)

        barrier_sem = pltpu.get_barrier_semaphore()
        for nbr in (left, right):
            pl.semaphore_signal(
                barrier_sem, inc=1,
                device_id=(nbr,), device_id_type=pl.DeviceIdType.MESH,
            )
        pl.semaphore_wait(barrier_sem, 2)

        def head_rdma(buf, s_slot, d_slot, h, send, recv, dev):
            return pltpu.make_async_remote_copy(
                src_ref=buf.at[s_slot, h], dst_ref=buf.at[d_slot, h],
                send_sem=send, recv_sem=recv,
                device_id=(dev,), device_id_type=pl.DeviceIdType.MESH,
            )

        f1k = [head_rdma(k_all, 0, 1, h, kf_send.at[0, h], kf_recv.at[0, h], right)
               for h in range(HQ)]
        f1v = [head_rdma(v_all, 0, 1, h, vf_send.at[0, h], vf_recv.at[0, h], right)
               for h in range(HQ)]
        f2k = [head_rdma(k_all, 1, 2, h, kf_send.at[1, h], kf_recv.at[1, h], right)
               for h in range(HQ)]
        f2v = [head_rdma(v_all, 1, 2, h, vf_send.at[1, h], vf_recv.at[1, h], right)
               for h in range(HQ)]
        b1k = [head_rdma(k_all, 0, 3, h, kb_send.at[h], kb_recv.at[h], left)
               for h in range(HQ)]
        b1v = [head_rdma(v_all, 0, 3, h, vb_send.at[h], vb_recv.at[h], left)
               for h in range(HQ)]

        for h in range(HQ):
            hs = slice(h * DH, (h + 1) * DH)
            k_all[0, h] = k_ref[:, hs]
            v_all[0, h] = v_ref[:, hs]
        for h in range(HQ):
            f1k[h].start()
            f1v[h].start()
        for h in range(HQ):
            b1k[h].start()
            b1v[h].start()

        q_buf[...] = (
            lax.dot_general(
                x_ref[...].astype(jnp.bfloat16),
                wq_ref[...].astype(jnp.bfloat16),
                (((1,), (0,)), ((), ())),
                preferred_element_type=jnp.float32,
            ) * Q_SCALE
        ).astype(jnp.bfloat16)

        r = lax.broadcasted_iota(jnp.int32, (S_PER, S_PER), 0)
        c = lax.broadcasted_iota(jnp.int32, (S_PER, S_PER), 1)
        bias_buf[...] = jnp.where(c // BLK <= r // BLK, 0.0, NEG).astype(
            jnp.bfloat16
        )

        def qk_exp(rows, hs, j, h, cols):
            s = lax.dot_general(
                q_buf[rows, hs], k_all[j, h, cols], (((1,), (1,)), ((), ())),
                preferred_element_type=jnp.float32,
            ).astype(jnp.bfloat16)
            return s

        def process(j, h):
            hs = slice(h * DH, (h + 1) * DH)
            p = jnp.exp2(qk_exp(slice(None), hs, j, h, slice(None)))
            lsum_buf[h] = lsum_buf[h] + jnp.sum(
                p, axis=1, keepdims=True, dtype=jnp.float32
            )
            acc_buf[:, hs] = acc_buf[:, hs] + lax.dot_general(
                p, v_all[j, h], (((1,), (0,)), ((), ())),
                preferred_element_type=jnp.float32,
            )

        lo = slice(0, HALF)
        for h in range(HQ):
            hs = slice(h * DH, (h + 1) * DH)
            pA = jnp.exp2(qk_exp(lo, hs, 0, h, lo) + bias_buf[lo, lo])
            lsum_buf[h, lo] = jnp.sum(pA, axis=1, keepdims=True,
                                      dtype=jnp.float32)
            acc_buf[lo, hs] = lax.dot_general(
                pA, v_all[0, h, lo], (((1,), (0,)), ((), ())),
                preferred_element_type=jnp.float32,
            )
            hi = slice(HALF, S_PER)
            pB = jnp.exp2(qk_exp(hi, hs, 0, h, slice(None)) + bias_buf[hi, :])
            lsum_buf[h, hi] = jnp.sum(pB, axis=1, keepdims=True,
                                      dtype=jnp.float32)
            acc_buf[hi, hs] = lax.dot_general(
                pB, v_all[0, h], (((1,), (0,)), ((), ())),
                preferred_element_type=jnp.float32,
            )

        for h in range(HQ):
            f1k[h].wait_recv()
            f2k[h].start()
            f1v[h].wait_recv()
            f2v[h].start()

            @pl.when(my >= 1)
            def _(h=h):
                process(1, h)

        for h in range(HQ):
            b1k[h].wait_recv()
            b1v[h].wait_recv()

            @pl.when(my >= 3)
            def _(h=h):
                process(3, h)

        for h in range(HQ):
            f2k[h].wait_recv()
            f2v[h].wait_recv()

            @pl.when(my >= 2)
            def _(h=h):
                process(2, h)

        for h in range(HQ):
            hs = slice(h * DH, (h + 1) * DH)
            ctx_buf[:, hs] = (
                acc_buf[:, hs] * (1.0 / lsum_buf[h])
            ).astype(jnp.bfloat16)
        out_ref[...] = lax.dot_general(
            ctx_buf[...], wo_ref[...].astype(jnp.bfloat16),
            (((1,), (0,)), ((), ())),
            preferred_element_type=jnp.float32,
        )

        for ds in (f1k, f1v, f2k, f2v, b1k, b1v):
            for d in ds:
                d.wait_send()

    out = pl.pallas_call(
        body,
        out_shape=jax.ShapeDtypeStruct((S_PER, D_MODEL), jnp.float32),
        in_specs=[pl.BlockSpec(memory_space=pltpu.VMEM)] * 5,
        out_specs=pl.BlockSpec(memory_space=pltpu.VMEM),
        scratch_shapes=[
            pltpu.VMEM((N_DEV, HQ, S_PER, DH), jnp.bfloat16),
            pltpu.VMEM((N_DEV, HQ, S_PER, DH), jnp.bfloat16),
            pltpu.VMEM((S_PER, D_MODEL), jnp.bfloat16),
            pltpu.VMEM((S_PER, S_PER), jnp.bfloat16),
            pltpu.VMEM((S_PER, D_MODEL), jnp.float32),
            pltpu.VMEM((S_PER, D_MODEL), jnp.bfloat16),
            pltpu.VMEM((HQ, S_PER, 1), jnp.float32),
            pltpu.SemaphoreType.DMA((2, HQ)),
            pltpu.SemaphoreType.DMA((2, HQ)),
            pltpu.SemaphoreType.DMA((2, HQ)),
            pltpu.SemaphoreType.DMA((2, HQ)),
            pltpu.SemaphoreType.DMA((HQ,)),
            pltpu.SemaphoreType.DMA((HQ,)),
            pltpu.SemaphoreType.DMA((HQ,)),
            pltpu.SemaphoreType.DMA((HQ,)),
        ],
        compiler_params=pltpu.CompilerParams(
            collective_id=0, vmem_limit_bytes=100 * 1024 * 1024
        ),
    )(x2, Wq, k2, v2, Wo)
    return out.reshape(1, S_PER, D_MODEL)


# device time: 105502 ns/iter; 1.1327x vs baseline; 1.1327x over previous
import jax
import jax.numpy as jnp
import math
from jax import lax
from jax.experimental import pallas as pl
from jax.experimental.pallas import tpu as pltpu

N_DEV = 4
S_PER = 1024
D_MODEL = 1024
HQ = 8
DH = 128
BLK = 64
SCALE = 0.08838834764831843
Q_SCALE = SCALE * math.log2(math.e)
NEG = -1e9
HALF = S_PER // 2


def kernel(x, Wq, K_ext, V_ext, Wo):
    x2 = x.reshape(S_PER, D_MODEL)
    k2 = K_ext.reshape(S_PER, HQ * DH).astype(jnp.bfloat16)
    v2 = V_ext.reshape(S_PER, HQ * DH).astype(jnp.bfloat16)

    def body(x_ref, wq_ref, k_ref, v_ref, wo_ref, out_ref,
             k_all, v_all, q_buf, bias_buf, acc_buf, ctx_buf, lsum_buf,
             kf_send, kf_recv, vf_send, vf_recv,
             kb_send, kb_recv, vb_send, vb_recv,
             kb2_send, kb2_recv, vb2_send, vb2_recv):
        my = lax.axis_index("i")
        left = lax.rem(my + (N_DEV - 1), N_DEV)
        right = lax.rem(my + 1, N_DEV)

        barrier_sem = pltpu.get_barrier_semaphore()
        for nbr in (left, right):
            pl.semaphore_signal(
                barrier_sem, inc=1,
                device_id=(nbr,), device_id_type=pl.DeviceIdType.MESH,
            )
        pl.semaphore_wait(barrier_sem, 2)

        def head_rdma(buf, s_slot, d_slot, h, send, recv, dev):
            return pltpu.make_async_remote_copy(
                src_ref=buf.at[s_slot, h], dst_ref=buf.at[d_slot, h],
                send_sem=send, recv_sem=recv,
                device_id=(dev,), device_id_type=pl.DeviceIdType.MESH,
            )

        f1k = [head_rdma(k_all, 0, 1, h, kf_send.at[0, h], kf_recv.at[0, h], right)
               for h in range(HQ)]
        f1v = [head_rdma(v_all, 0, 1, h, vf_send.at[0, h], vf_recv.at[0, h], right)
               for h in range(HQ)]
        f2k = [head_rdma(k_all, 1, 2, h, kf_send.at[1, h], kf_recv.at[1, h], right)
               for h in range(HQ // 2)]
        f2v = [head_rdma(v_all, 1, 2, h, vf_send.at[1, h], vf_recv.at[1, h], right)
               for h in range(HQ // 2)]
        b2k = [head_rdma(k_all, 3, 2, h, kb2_send.at[h - 4], kb2_recv.at[h - 4], left)
               for h in range(HQ // 2, HQ)]
        b2v = [head_rdma(v_all, 3, 2, h, vb2_send.at[h - 4], vb2_recv.at[h - 4], left)
               for h in range(HQ // 2, HQ)]
        b1k = [head_rdma(k_all, 0, 3, h, kb_send.at[h], kb_recv.at[h], left)
               for h in range(HQ)]
        b1v = [head_rdma(v_all, 0, 3, h, vb_send.at[h], vb_recv.at[h], left)
               for h in range(HQ)]

        scope = jax.named_scope
        for h in range(HQ):
            hs = slice(h * DH, (h + 1) * DH)
            k_all[0, h] = k_ref[:, hs]
            v_all[0, h] = v_ref[:, hs]
        for h in range(HQ):
            f1k[h].start()
            f1v[h].start()
        for h in range(HQ):
            b1k[h].start()
            b1v[h].start()

        sc_q = scope("phase_q"); sc_q.__enter__()
        q_buf[...] = (
            lax.dot_general(
                x_ref[...].astype(jnp.bfloat16),
                wq_ref[...].astype(jnp.bfloat16),
                (((1,), (0,)), ((), ())),
                preferred_element_type=jnp.float32,
            ) * Q_SCALE
        ).astype(jnp.bfloat16)

        r = lax.broadcasted_iota(jnp.int32, (S_PER, S_PER), 0)
        c = lax.broadcasted_iota(jnp.int32, (S_PER, S_PER), 1)
        bias_buf[...] = jnp.where(c // BLK <= r // BLK, 0.0, NEG).astype(
            jnp.bfloat16
        )

        sc_q.__exit__(None, None, None)

        def qk_exp(rows, hs, j, h, cols):
            s = lax.dot_general(
                q_buf[rows, hs], k_all[j, h, cols], (((1,), (1,)), ((), ())),
                preferred_element_type=jnp.float32,
            ).astype(jnp.bfloat16)
            return s

        def process(j, h):
            hs = slice(h * DH, (h + 1) * DH)
            p = jnp.exp2(qk_exp(slice(None), hs, j, h, slice(None)))
            lsum_buf[h] = lsum_buf[h] + jnp.sum(
                p, axis=1, keepdims=True, dtype=jnp.float32
            )
            acc_buf[:, hs] = acc_buf[:, hs] + lax.dot_general(
                p, v_all[j, h], (((1,), (0,)), ((), ())),
                preferred_element_type=jnp.float32,
            )

        sc0 = scope("phase_chunk0"); sc0.__enter__()
        lo = slice(0, HALF)
        for h in range(HQ):
            hs = slice(h * DH, (h + 1) * DH)
            pA = jnp.exp2(qk_exp(lo, hs, 0, h, lo) + bias_buf[lo, lo])
            lsum_buf[h, lo] = jnp.sum(pA, axis=1, keepdims=True,
                                      dtype=jnp.float32)
            acc_buf[lo, hs] = lax.dot_general(
                pA, v_all[0, h, lo], (((1,), (0,)), ((), ())),
                preferred_element_type=jnp.float32,
            )
            hi = slice(HALF, S_PER)
            pB = jnp.exp2(qk_exp(hi, hs, 0, h, slice(None)) + bias_buf[hi, :])
            lsum_buf[h, hi] = jnp.sum(pB, axis=1, keepdims=True,
                                      dtype=jnp.float32)
            acc_buf[hi, hs] = lax.dot_general(
                pB, v_all[0, h], (((1,), (0,)), ((), ())),
                preferred_element_type=jnp.float32,
            )

        sc0.__exit__(None, None, None)
        sc1 = scope("phase_slot1"); sc1.__enter__()
        for h in range(HQ):
            f1k[h].wait_recv()
            if h < HQ // 2:
                f2k[h].start()
            f1v[h].wait_recv()
            if h < HQ // 2:
                f2v[h].start()

            @pl.when(my >= 1)
            def _(h=h):
                process(1, h)

        sc1.__exit__(None, None, None)
        sc3 = scope("phase_slot3"); sc3.__enter__()
        for h in range(HQ):
            b1k[h].wait_recv()
            b1v[h].wait_recv()
            if h >= HQ // 2:
                b2k[h - 4].start()
                b2v[h - 4].start()

            @pl.when(my >= 3)
            def _(h=h):
                process(3, h)

        sc3.__exit__(None, None, None)
        sc2 = scope("phase_slot2"); sc2.__enter__()
        for h in range(HQ):
            if h < HQ // 2:
                f2k[h].wait_recv()
                f2v[h].wait_recv()
            else:
                b2k[h - 4].wait_recv()
                b2v[h - 4].wait_recv()

            @pl.when(my >= 2)
            def _(h=h):
                process(2, h)

        sc2.__exit__(None, None, None)
        scf = scope("phase_final"); scf.__enter__()
        for h in range(HQ):
            hs = slice(h * DH, (h + 1) * DH)
            ctx_buf[:, hs] = (
                acc_buf[:, hs] * (1.0 / lsum_buf[h])
            ).astype(jnp.bfloat16)
        out_ref[...] = lax.dot_general(
            ctx_buf[...], wo_ref[...].astype(jnp.bfloat16),
            (((1,), (0,)), ((), ())),
            preferred_element_type=jnp.float32,
        )

        scf.__exit__(None, None, None)
        for ds in (f1k, f1v, f2k, f2v, b1k, b1v, b2k, b2v):
            for d in ds:
                d.wait_send()

    out = pl.pallas_call(
        body,
        out_shape=jax.ShapeDtypeStruct((S_PER, D_MODEL), jnp.float32),
        in_specs=[pl.BlockSpec(memory_space=pltpu.VMEM)] * 5,
        out_specs=pl.BlockSpec(memory_space=pltpu.VMEM),
        scratch_shapes=[
            pltpu.VMEM((N_DEV, HQ, S_PER, DH), jnp.bfloat16),
            pltpu.VMEM((N_DEV, HQ, S_PER, DH), jnp.bfloat16),
            pltpu.VMEM((S_PER, D_MODEL), jnp.bfloat16),
            pltpu.VMEM((S_PER, S_PER), jnp.bfloat16),
            pltpu.VMEM((S_PER, D_MODEL), jnp.float32),
            pltpu.VMEM((S_PER, D_MODEL), jnp.bfloat16),
            pltpu.VMEM((HQ, S_PER, 1), jnp.float32),
            pltpu.SemaphoreType.DMA((2, HQ)),
            pltpu.SemaphoreType.DMA((2, HQ)),
            pltpu.SemaphoreType.DMA((2, HQ)),
            pltpu.SemaphoreType.DMA((2, HQ)),
            pltpu.SemaphoreType.DMA((HQ,)),
            pltpu.SemaphoreType.DMA((HQ,)),
            pltpu.SemaphoreType.DMA((HQ,)),
            pltpu.SemaphoreType.DMA((HQ,)),
            pltpu.SemaphoreType.DMA((HQ // 2,)),
            pltpu.SemaphoreType.DMA((HQ // 2,)),
            pltpu.SemaphoreType.DMA((HQ // 2,)),
            pltpu.SemaphoreType.DMA((HQ // 2,)),
        ],
        compiler_params=pltpu.CompilerParams(
            collective_id=0, vmem_limit_bytes=100 * 1024 * 1024
        ),
    )(x2, Wq, k2, v2, Wo)
    return out.reshape(1, S_PER, D_MODEL)
